# Initial kernel scaffold; baseline (speedup 1.0000x reference)
import jax
import jax.numpy as jnp
from jax.experimental import pallas as pl


def kernel(encoder_output, opt, opt_len, embed_table, w_ih_l0, w_hh_l0, b_ih_l0, b_hh_l0, w_ih_l1, w_hh_l1, b_ih_l1, b_hh_l1):
    raise NotImplementedError("write your pallas kernel here")


# speedup vs baseline: 2.9074x; 2.9074x over previous
import functools

import jax
import jax.numpy as jnp
from jax import lax
from jax.experimental import pallas as pl
from jax.experimental.pallas import tpu as pltpu
from jax.experimental.pallas import tpu_sc as plsc

VOCAB = 11322
EMBED = 300
DP = 384
HIDDEN = 512
G4 = 4 * HIDDEN
B, R, O, L = 4, 10, 100, 20
N = B * R * O
NP = 4096
NB = 256
NTOK = L * NP

_NC, _NS = 2, 16
_NW = _NC * _NS
_PER_W = NTOK // _NW
_CHUNK = 128


def _sc_gather(ids_flat, table_pad):
    mesh = plsc.VectorSubcoreMesh(core_axis_name="c", subcore_axis_name="s")

    @functools.partial(
        pl.kernel,
        mesh=mesh,
        out_type=jax.ShapeDtypeStruct((NTOK, DP), jnp.float32),
        scratch_types=[
            pltpu.VMEM((_PER_W,), jnp.int32),
            pltpu.VMEM((_CHUNK, DP), jnp.float32),
            pltpu.SemaphoreType.DMA,
        ],
    )
    def gather_k(ids_hbm, table_hbm, out_hbm, idx_v, rows_v, sem):
        wid = lax.axis_index("s") * _NC + lax.axis_index("c")
        base = wid * _PER_W
        pltpu.sync_copy(ids_hbm.at[pl.ds(base, _PER_W)], idx_v)

        def chunk(k, carry):
            off = k * _CHUNK
            pltpu.async_copy(
                table_hbm.at[idx_v.at[pl.ds(off, _CHUNK)]], rows_v, sem
            ).wait()
            pltpu.sync_copy(rows_v, out_hbm.at[pl.ds(base + off, _CHUNK)])
            return carry

        lax.fori_loop(0, _PER_W // _CHUNK, chunk, 0)

    return gather_k(ids_flat, table_pad)


def _lstm_body(emb_ref, lens_ref, enc_ref, wih0, whh0, b0, wih1, whh1, b1,
               out_ref, h0, c0, h1, c1):
    zeros = jnp.zeros((NB, HIDDEN), jnp.float32)
    h0[...] = zeros
    c0[...] = zeros
    h1[...] = zeros
    c1[...] = zeros

    def step(t, carry):
        x = emb_ref[t]
        m = lens_ref[...] > t

        g = (jnp.dot(x, wih0[...], preferred_element_type=jnp.float32)
             + jnp.dot(h0[...], whh0[...], preferred_element_type=jnp.float32)
             + b0[...])
        ig = jax.nn.sigmoid(g[:, 0:HIDDEN])
        fg = jax.nn.sigmoid(g[:, HIDDEN:2 * HIDDEN])
        gg = jnp.tanh(g[:, 2 * HIDDEN:3 * HIDDEN])
        og = jax.nn.sigmoid(g[:, 3 * HIDDEN:4 * HIDDEN])
        cn = fg * c0[...] + ig * gg
        hn = og * jnp.tanh(cn)
        h0[...] = jnp.where(m, hn, h0[...])
        c0[...] = jnp.where(m, cn, c0[...])

        g = (jnp.dot(h0[...], wih1[...], preferred_element_type=jnp.float32)
             + jnp.dot(h1[...], whh1[...], preferred_element_type=jnp.float32)
             + b1[...])
        ig = jax.nn.sigmoid(g[:, 0:HIDDEN])
        fg = jax.nn.sigmoid(g[:, HIDDEN:2 * HIDDEN])
        gg = jnp.tanh(g[:, 2 * HIDDEN:3 * HIDDEN])
        og = jax.nn.sigmoid(g[:, 3 * HIDDEN:4 * HIDDEN])
        cn = fg * c1[...] + ig * gg
        hn = og * jnp.tanh(cn)
        h1[...] = jnp.where(m, hn, h1[...])
        c1[...] = jnp.where(m, cn, c1[...])
        return carry

    lax.fori_loop(0, L, step, 0)
    out_ref[...] = jnp.sum(h1[...] * enc_ref[...], axis=1, keepdims=True)


def _tc_lstm(emb, lens, enc, wih0, whh0, b0, wih1, whh1, b1):
    grid = (NP // NB,)
    return pl.pallas_call(
        _lstm_body,
        grid=grid,
        in_specs=[
            pl.BlockSpec((L, NB, DP), lambda i: (0, i, 0)),
            pl.BlockSpec((NB, 1), lambda i: (i, 0)),
            pl.BlockSpec((NB, HIDDEN), lambda i: (i, 0)),
            pl.BlockSpec((DP, G4), lambda i: (0, 0)),
            pl.BlockSpec((HIDDEN, G4), lambda i: (0, 0)),
            pl.BlockSpec((1, G4), lambda i: (0, 0)),
            pl.BlockSpec((HIDDEN, G4), lambda i: (0, 0)),
            pl.BlockSpec((HIDDEN, G4), lambda i: (0, 0)),
            pl.BlockSpec((1, G4), lambda i: (0, 0)),
        ],
        out_specs=pl.BlockSpec((NB, 1), lambda i: (i, 0)),
        out_shape=jax.ShapeDtypeStruct((NP, 1), jnp.float32),
        scratch_shapes=[pltpu.VMEM((NB, HIDDEN), jnp.float32)] * 4,
        compiler_params=pltpu.CompilerParams(
            dimension_semantics=("arbitrary",),
        ),
    )(emb, lens, enc, wih0, whh0, b0, wih1, whh1, b1)


def kernel(encoder_output, opt, opt_len, embed_table,
           w_ih_l0, w_hh_l0, b_ih_l0, b_hh_l0,
           w_ih_l1, w_hh_l1, b_ih_l1, b_hh_l1):
    ids = opt.reshape(N, L).astype(jnp.int32).T
    ids = jnp.pad(ids, ((0, 0), (0, NP - N))).reshape(NTOK)
    lens = jnp.pad(opt_len.reshape(N).astype(jnp.int32), (0, NP - N))
    lens = lens.reshape(NP, 1)
    table = jnp.pad(embed_table, ((0, 0), (0, DP - EMBED)))
    table = table.at[0].set(0.0)
    enc = jnp.broadcast_to(
        encoder_output[:, :, None, :], (B, R, O, HIDDEN)).reshape(N, HIDDEN)
    enc = jnp.pad(enc, ((0, NP - N), (0, 0)))
    wih0 = jnp.pad(w_ih_l0.T, ((0, DP - EMBED), (0, 0)))
    whh0 = w_hh_l0.T
    wih1 = w_ih_l1.T
    whh1 = w_hh_l1.T
    b0 = (b_ih_l0 + b_hh_l0).reshape(1, G4)
    b1 = (b_ih_l1 + b_hh_l1).reshape(1, G4)

    emb = _sc_gather(ids, table).reshape(L, NP, DP)

    scores = _tc_lstm(emb, lens, enc, wih0, whh0, b0, wih1, whh1, b1)
    return scores[:N, 0].reshape(B, R, O)


# speedup vs baseline: 4.2081x; 1.4474x over previous
import functools

import jax
import jax.numpy as jnp
from jax import lax
from jax.experimental import pallas as pl
from jax.experimental.pallas import tpu as pltpu
from jax.experimental.pallas import tpu_sc as plsc

VOCAB = 11322
EMBED = 300
DP = 384
HIDDEN = 512
G4 = 4 * HIDDEN
B, R, O, L = 4, 10, 100, 20
N = B * R * O
NP = 4096
NB = 256
NTOK = L * NP

_NC, _NS = 2, 16
_NW = _NC * _NS
_PER_W = NTOK // _NW
_CHUNK = 128


def _sc_gather(ids_flat, table_pad):
    mesh = plsc.VectorSubcoreMesh(core_axis_name="c", subcore_axis_name="s")

    @functools.partial(
        pl.kernel,
        mesh=mesh,
        out_type=jax.ShapeDtypeStruct((NTOK, DP), jnp.float32),
        scratch_types=[
            pltpu.VMEM((_PER_W,), jnp.int32),
            pltpu.VMEM((_CHUNK, DP), jnp.float32),
            pltpu.SemaphoreType.DMA,
        ],
    )
    def gather_k(ids_hbm, table_hbm, out_hbm, idx_v, rows_v, sem):
        wid = lax.axis_index("s") * _NC + lax.axis_index("c")
        base = wid * _PER_W
        pltpu.sync_copy(ids_hbm.at[pl.ds(base, _PER_W)], idx_v)

        def chunk(k, carry):
            off = k * _CHUNK
            pltpu.async_copy(
                table_hbm.at[idx_v.at[pl.ds(off, _CHUNK)]], rows_v, sem
            ).wait()
            pltpu.sync_copy(rows_v, out_hbm.at[pl.ds(base + off, _CHUNK)])
            return carry

        lax.fori_loop(0, _PER_W // _CHUNK, chunk, 0)

    return gather_k(ids_flat, table_pad)


def _lstm_body(tmax_ref, emb_ref, lens_ref, enc_ref, wih0, whh0, b0, wih1,
               whh1, b1, out_ref, h0, c0, h1, c1):
    zeros = jnp.zeros((NB, HIDDEN), jnp.float32)
    h0[...] = zeros
    c0[...] = zeros
    h1[...] = zeros
    c1[...] = zeros
    t_hi = tmax_ref[pl.program_id(0)]

    def step(t, carry):
        x = emb_ref[t].astype(jnp.bfloat16)
        m = lens_ref[...] > t

        g = (jnp.dot(x, wih0[...], preferred_element_type=jnp.float32)
             + jnp.dot(h0[...].astype(jnp.bfloat16), whh0[...],
                       preferred_element_type=jnp.float32)
             + b0[...])
        ig = jax.nn.sigmoid(g[:, 0:HIDDEN])
        fg = jax.nn.sigmoid(g[:, HIDDEN:2 * HIDDEN])
        gg = jnp.tanh(g[:, 2 * HIDDEN:3 * HIDDEN])
        og = jax.nn.sigmoid(g[:, 3 * HIDDEN:4 * HIDDEN])
        cn = fg * c0[...] + ig * gg
        hn = og * jnp.tanh(cn)
        h0[...] = jnp.where(m, hn, h0[...])
        c0[...] = jnp.where(m, cn, c0[...])

        g = (jnp.dot(h0[...].astype(jnp.bfloat16), wih1[...],
                     preferred_element_type=jnp.float32)
             + jnp.dot(h1[...].astype(jnp.bfloat16), whh1[...],
                       preferred_element_type=jnp.float32)
             + b1[...])
        ig = jax.nn.sigmoid(g[:, 0:HIDDEN])
        fg = jax.nn.sigmoid(g[:, HIDDEN:2 * HIDDEN])
        gg = jnp.tanh(g[:, 2 * HIDDEN:3 * HIDDEN])
        og = jax.nn.sigmoid(g[:, 3 * HIDDEN:4 * HIDDEN])
        cn = fg * c1[...] + ig * gg
        hn = og * jnp.tanh(cn)
        h1[...] = jnp.where(m, hn, h1[...])
        c1[...] = jnp.where(m, cn, c1[...])
        return carry

    lax.fori_loop(0, t_hi, step, 0)
    out_ref[...] = jnp.sum(h1[...] * enc_ref[...], axis=1, keepdims=True)


def _tc_lstm(tmax, emb, lens, enc, wih0, whh0, b0, wih1, whh1, b1):
    grid = (NP // NB,)
    grid_spec = pltpu.PrefetchScalarGridSpec(
        num_scalar_prefetch=1,
        grid=grid,
        in_specs=[
            pl.BlockSpec((L, NB, DP), lambda i, *_: (0, i, 0)),
            pl.BlockSpec((NB, 1), lambda i, *_: (i, 0)),
            pl.BlockSpec((NB, HIDDEN), lambda i, *_: (i, 0)),
            pl.BlockSpec((DP, G4), lambda i, *_: (0, 0)),
            pl.BlockSpec((HIDDEN, G4), lambda i, *_: (0, 0)),
            pl.BlockSpec((1, G4), lambda i, *_: (0, 0)),
            pl.BlockSpec((HIDDEN, G4), lambda i, *_: (0, 0)),
            pl.BlockSpec((HIDDEN, G4), lambda i, *_: (0, 0)),
            pl.BlockSpec((1, G4), lambda i, *_: (0, 0)),
        ],
        out_specs=pl.BlockSpec((NB, 1), lambda i, *_: (i, 0)),
        scratch_shapes=[pltpu.VMEM((NB, HIDDEN), jnp.float32)] * 4,
    )
    return pl.pallas_call(
        _lstm_body,
        grid_spec=grid_spec,
        out_shape=jax.ShapeDtypeStruct((NP, 1), jnp.float32),
        compiler_params=pltpu.CompilerParams(
            dimension_semantics=("arbitrary",),
        ),
    )(tmax, emb, lens, enc, wih0, whh0, b0, wih1, whh1, b1)


def kernel(encoder_output, opt, opt_len, embed_table,
           w_ih_l0, w_hh_l0, b_ih_l0, b_hh_l0,
           w_ih_l1, w_hh_l1, b_ih_l1, b_hh_l1):
    lens0 = opt_len.reshape(N).astype(jnp.int32)
    perm = jnp.argsort(-lens0)
    lens_s = lens0[perm]
    ids = opt.reshape(N, L).astype(jnp.int32)[perm].T
    ids = jnp.pad(ids, ((0, 0), (0, NP - N))).reshape(NTOK)
    lens_p = jnp.pad(lens_s, (0, NP - N))
    lens = lens_p.reshape(NP, 1)
    tmax = lens_p[0::NB]
    table = jnp.pad(embed_table, ((0, 0), (0, DP - EMBED)))
    table = table.at[0].set(0.0)
    enc = jnp.broadcast_to(
        encoder_output[:, :, None, :], (B, R, O, HIDDEN)).reshape(N, HIDDEN)
    enc = jnp.pad(enc[perm], ((0, NP - N), (0, 0)))
    bf16 = jnp.bfloat16
    wih0 = jnp.pad(w_ih_l0.T, ((0, DP - EMBED), (0, 0))).astype(bf16)
    whh0 = w_hh_l0.T.astype(bf16)
    wih1 = w_ih_l1.T.astype(bf16)
    whh1 = w_hh_l1.T.astype(bf16)
    b0 = (b_ih_l0 + b_hh_l0).reshape(1, G4)
    b1 = (b_ih_l1 + b_hh_l1).reshape(1, G4)

    emb = _sc_gather(ids, table).reshape(L, NP, DP)

    scores = _tc_lstm(tmax, emb, lens, enc, wih0, whh0, b0, wih1, whh1, b1)
    out = jnp.zeros((N,), jnp.float32).at[perm].set(scores[:N, 0])
    return out.reshape(B, R, O)
